# baseline (speedup 1.0000x reference)
import functools

import jax
import jax.numpy as jnp
from jax import lax
from jax.experimental import pallas as pl
from jax.experimental.pallas import tpu as pltpu
from jax.experimental.pallas import tpu_sc as plsc

DIM = 16
PDIM = DIM // 2
N_ROWS = 1000000
BATCH = 16384
NUM_CORES = 2
NUM_SUBCORES = 16
LANES = 16
NUM_WORKERS = NUM_CORES * NUM_SUBCORES
BPW = BATCH // NUM_WORKERS
IDX_CHUNK = 128
N_CHUNKS = BPW // IDX_CHUNK


@functools.partial(
    pl.kernel,
    out_type=jax.ShapeDtypeStruct((BATCH,), jnp.float32),
    mesh=plsc.VectorSubcoreMesh(core_axis_name="c", subcore_axis_name="s"),
    compiler_params=pltpu.CompilerParams(use_tc_tiling_on_sc=False,
                                         needs_layout_passes=False),
    scratch_types=[
        pltpu.VMEM((BPW,), jnp.int32),
        pltpu.VMEM((BPW,), jnp.int32),
        pltpu.VMEM((N_CHUNKS, IDX_CHUNK), jnp.int32),
        pltpu.VMEM((N_CHUNKS, IDX_CHUNK), jnp.int32),
        pltpu.VMEM((BPW, DIM), jnp.int32),
        pltpu.VMEM((BPW, DIM), jnp.int32),
        pltpu.VMEM((BPW,), jnp.float32),
        pltpu.SemaphoreType.DMA,
        pltpu.SemaphoreType.DMA,
    ],
)
def _mf_kernel(idx0_hbm, idx1_hbm, u_hbm, v_hbm, out_hbm,
               idx0_v, idx1_v, q0_v, q1_v, upack, vpack, outv, sem_u, sem_v):
    wid = lax.axis_index("s") * NUM_CORES + lax.axis_index("c")
    base = wid * BPW

    pltpu.sync_copy(idx0_hbm.at[pl.ds(base, BPW)], idx0_v)
    pltpu.sync_copy(idx1_hbm.at[pl.ds(base, BPW)], idx1_v)

    lanes = lax.iota(jnp.int32, LANES)

    def build_q(i, carry):
        o16 = pl.multiple_of(i * LANES, LANES)
        r0 = idx0_v[pl.ds(o16, LANES)]
        r1 = idx1_v[pl.ds(o16, LANES)]
        c = o16 // IDX_CHUNK
        q0_v[c, pl.ds(o16 % IDX_CHUNK, LANES)] = jnp.right_shift(r0, 1)
        q1_v[c, pl.ds(o16 % IDX_CHUNK, LANES)] = jnp.right_shift(r1, 1)
        return carry

    for i in range(BPW // LANES):
        build_q(i, 0)

    copies = []
    for j in range(N_CHUNKS):
        copies.append(pltpu.async_copy(
            u_hbm.at[q0_v.at[j]],
            upack.at[pl.ds(j * IDX_CHUNK, IDX_CHUNK)], sem_u))
        copies.append(pltpu.async_copy(
            v_hbm.at[q1_v.at[j]],
            vpack.at[pl.ds(j * IDX_CHUNK, IDX_CHUNK)], sem_v))
    for cp in copies:
        cp.wait()

    lane0 = jnp.bitwise_and(lanes, 0)
    himask = lane0 - 65536

    def to_f32(bits):
        return lax.bitcast_convert_type(bits, jnp.float32)

    def group(g, carry):
        gbase = pl.multiple_of(g * LANES, LANES)
        r0 = idx0_v[pl.ds(gbase, LANES)]
        r1 = idx1_v[pl.ds(gbase, LANES)]
        col0 = jnp.left_shift(jnp.bitwise_and(r0, 1), 3)
        col1 = jnp.left_shift(jnp.bitwise_and(r1, 1), 3)
        rows = gbase + lanes
        acc = jnp.zeros((LANES,), jnp.float32)
        for w in range(PDIM):
            wu = plsc.load_gather(upack, [rows, col0 + w])
            wv = plsc.load_gather(vpack, [rows, col1 + w])
            u_e = to_f32(jnp.left_shift(wu, 16))
            u_o = to_f32(jnp.bitwise_and(wu, himask))
            v_e = to_f32(jnp.left_shift(wv, 16))
            v_o = to_f32(jnp.bitwise_and(wv, himask))
            acc = acc + u_e * v_e + u_o * v_o
        outv[pl.ds(gbase, LANES)] = acc
        return carry

    lax.fori_loop(0, BPW // LANES, group, 0)

    pltpu.sync_copy(outv, out_hbm.at[pl.ds(base, BPW)])


def kernel(x, U, V):
    xi = x.astype(jnp.int32)
    ubits = lax.bitcast_convert_type(
        U.astype(jnp.bfloat16).reshape(N_ROWS // 2, DIM, 2), jnp.int32)
    vbits = lax.bitcast_convert_type(
        V.astype(jnp.bfloat16).reshape(N_ROWS // 2, DIM, 2), jnp.int32)
    return _mf_kernel(xi[:, 0], xi[:, 1], ubits, vbits)
